# baseline (speedup 1.0000x reference)
import jax
import jax.numpy as jnp
from jax.experimental import pallas as pl

_N = 8192
_HID = 32
_OSIZE = 128
_K = 30
_EPS = 0.1
_R4 = 32


def _p4_body(emb_ref, embt_ref, o_ref):
    sim = jnp.dot(emb_ref[...], embt_ref[...],
                  preferred_element_type=jnp.float32)
    R, NC = sim.shape
    cand = sim > _EPS
    work = jnp.where(cand, sim, -1.0)

    def cond(st):
        _, _, removed = st
        return jnp.any(removed < (_K + 1))

    def body(st):
        w, t, removed = st
        m = jnp.max(w, axis=1, keepdims=True)
        cnt = jnp.sum((w == m).astype(jnp.int32), axis=1, keepdims=True)
        active = removed < (_K + 1)
        t = jnp.where(active, m, t)
        removed = removed + jnp.where(active, cnt, 0)
        w = jnp.where(w == m, -1.0, w)
        return (w, t, removed)

    t0 = jnp.full((R, 1), 2.0, jnp.float32)
    r0 = jnp.zeros((R, 1), jnp.int32)
    _, t, _ = jax.lax.while_loop(cond, body, (work, t0, r0))

    gt = cand & (sim > t)
    c_gt = jnp.sum(gt.astype(jnp.int32), axis=1, keepdims=True)
    m_allow = (_K + 1) - c_gt
    eq = cand & (sim == t)
    eqf = eq.astype(jnp.float32)
    eq2 = eqf.reshape(R * (NC // 128), 128)
    i0 = jax.lax.broadcasted_iota(jnp.int32, (128, 128), 0)
    i1 = jax.lax.broadcasted_iota(jnp.int32, (128, 128), 1)
    tri = (i0 <= i1).astype(jnp.float32)
    within = jnp.dot(eq2, tri, preferred_element_type=jnp.float32)
    ctot = within[:, 127:128].reshape(R, NC // 128)
    j0 = jax.lax.broadcasted_iota(jnp.int32, (NC // 128, NC // 128), 0)
    j1 = jax.lax.broadcasted_iota(jnp.int32, (NC // 128, NC // 128), 1)
    stri = (j0 < j1).astype(jnp.float32)
    offs = jnp.dot(ctot, stri, preferred_element_type=jnp.float32)
    prefix = (within.reshape(R, NC // 128, 128)
              + offs[:, :, None]).reshape(R, NC)
    keep = gt | (eq & (prefix <= m_allow.astype(jnp.float32)))
    o_ref[...] = jnp.where(keep, sim, 0.0)


def _similarity_topk(emb, embt):
    return pl.pallas_call(
        _p4_body,
        grid=(_N // _R4,),
        in_specs=[
            pl.BlockSpec((_R4, _OSIZE), lambda i: (i, 0)),
            pl.BlockSpec((_OSIZE, _N), lambda i: (0, 0)),
        ],
        out_specs=pl.BlockSpec((_R4, _N), lambda i: (i, 0)),
        out_shape=jax.ShapeDtypeStruct((_N, _N), jnp.float32),
    )(emb, embt)


def kernel(features, adj, W1a, b1a, W2a, b2a, W1b, b1b, W2b, b2b):
    h = jax.nn.relu(features @ W1a + b1a)
    h = adj @ h
    h = h @ W2a + b2a
    h = jax.nn.relu(h)
    h2 = jax.nn.relu(h @ W1b + b1b)
    g2 = adj @ h2
    e = g2 @ W2b + b2b
    norm = jnp.linalg.norm(e, axis=1, keepdims=True)
    emb = e / jnp.maximum(norm, 1e-12)
    embt = emb.T
    return _similarity_topk(emb, embt)


# speedup vs baseline: 1.4016x; 1.4016x over previous
import jax
import jax.numpy as jnp
from jax.experimental import pallas as pl

_N = 8192
_HID = 32
_OSIZE = 128
_K = 30
_EPS = 0.1
_R4 = 64


def _p4_body(emb_ref, embt_ref, o_ref):
    sim = jnp.dot(emb_ref[...], embt_ref[...],
                  preferred_element_type=jnp.float32)
    R, NC = sim.shape
    cand = sim > _EPS
    work = jnp.where(cand, sim, -1.0)

    def cond(st):
        _, removed = st
        return jnp.any(removed < (_K + 1))

    def body(st):
        t, removed = st
        m = jnp.max(jnp.where(work < t, work, -1.0), axis=1, keepdims=True)
        cum = jnp.sum((work >= m).astype(jnp.int32), axis=1, keepdims=True)
        active = removed < (_K + 1)
        t = jnp.where(active, m, t)
        removed = jnp.where(active, cum, removed)
        return (t, removed)

    t0 = jnp.full((R, 1), 2.0, jnp.float32)
    r0 = jnp.zeros((R, 1), jnp.int32)
    t, _ = jax.lax.while_loop(cond, body, (t0, r0))

    gt = cand & (sim > t)
    c_gt = jnp.sum(gt.astype(jnp.int32), axis=1, keepdims=True)
    m_allow = (_K + 1) - c_gt
    eq = cand & (sim == t)
    eqf = eq.astype(jnp.float32)
    eq2 = eqf.reshape(R * (NC // 128), 128)
    i0 = jax.lax.broadcasted_iota(jnp.int32, (128, 128), 0)
    i1 = jax.lax.broadcasted_iota(jnp.int32, (128, 128), 1)
    tri = (i0 <= i1).astype(jnp.float32)
    within = jnp.dot(eq2, tri, preferred_element_type=jnp.float32)
    ctot = within[:, 127:128].reshape(R, NC // 128)
    j0 = jax.lax.broadcasted_iota(jnp.int32, (NC // 128, NC // 128), 0)
    j1 = jax.lax.broadcasted_iota(jnp.int32, (NC // 128, NC // 128), 1)
    stri = (j0 < j1).astype(jnp.float32)
    offs = jnp.dot(ctot, stri, preferred_element_type=jnp.float32)
    prefix = (within.reshape(R, NC // 128, 128)
              + offs[:, :, None]).reshape(R, NC)
    keep = gt | (eq & (prefix <= m_allow.astype(jnp.float32)))
    o_ref[...] = jnp.where(keep, sim, 0.0)


def _similarity_topk(emb, embt):
    return pl.pallas_call(
        _p4_body,
        grid=(_N // _R4,),
        in_specs=[
            pl.BlockSpec((_R4, _OSIZE), lambda i: (i, 0)),
            pl.BlockSpec((_OSIZE, _N), lambda i: (0, 0)),
        ],
        out_specs=pl.BlockSpec((_R4, _N), lambda i: (i, 0)),
        out_shape=jax.ShapeDtypeStruct((_N, _N), jnp.float32),
    )(emb, embt)


def kernel(features, adj, W1a, b1a, W2a, b2a, W1b, b1b, W2b, b2b):
    h = jax.nn.relu(features @ W1a + b1a)
    h = adj @ h
    h = h @ W2a + b2a
    h = jax.nn.relu(h)
    h2 = jax.nn.relu(h @ W1b + b1b)
    g2 = adj @ h2
    e = g2 @ W2b + b2b
    norm = jnp.linalg.norm(e, axis=1, keepdims=True)
    emb = e / jnp.maximum(norm, 1e-12)
    embt = emb.T
    return _similarity_topk(emb, embt)


# speedup vs baseline: 1.6072x; 1.1467x over previous
import jax
import jax.numpy as jnp
from jax.experimental import pallas as pl

_N = 8192
_HID = 32
_OSIZE = 128
_K = 30
_EPS = 0.1
_R4 = 128


def _p4_body(emb_ref, embt_ref, o_ref):
    sim = jnp.dot(emb_ref[...], embt_ref[...],
                  preferred_element_type=jnp.float32)
    R, NC = sim.shape
    cand = sim > _EPS
    work = jnp.where(cand, sim, -1.0)

    def cond(st):
        _, removed = st
        return jnp.any(removed < (_K + 1))

    def body(st):
        t, removed = st
        m = jnp.max(jnp.where(work < t, work, -1.0), axis=1, keepdims=True)
        cum = jnp.sum((work >= m).astype(jnp.int32), axis=1, keepdims=True)
        active = removed < (_K + 1)
        t = jnp.where(active, m, t)
        removed = jnp.where(active, cum, removed)
        return (t, removed)

    t0 = jnp.full((R, 1), 2.0, jnp.float32)
    r0 = jnp.zeros((R, 1), jnp.int32)
    t, _ = jax.lax.while_loop(cond, body, (t0, r0))

    gt = work > t
    c_gt = jnp.sum(gt.astype(jnp.int32), axis=1, keepdims=True)
    m_allow = (_K + 1) - c_gt
    eq = cand & (sim == t)
    eqf = eq.astype(jnp.float32)
    eq2 = eqf.reshape(R * (NC // 128), 128)
    i0 = jax.lax.broadcasted_iota(jnp.int32, (128, 128), 0)
    i1 = jax.lax.broadcasted_iota(jnp.int32, (128, 128), 1)
    tri = (i0 <= i1).astype(jnp.float32)
    within = jnp.dot(eq2, tri, preferred_element_type=jnp.float32)
    ctot = within[:, 127:128].reshape(R, NC // 128)
    j0 = jax.lax.broadcasted_iota(jnp.int32, (NC // 128, NC // 128), 0)
    j1 = jax.lax.broadcasted_iota(jnp.int32, (NC // 128, NC // 128), 1)
    stri = (j0 < j1).astype(jnp.float32)
    offs = jnp.dot(ctot, stri, preferred_element_type=jnp.float32)
    prefix = (within.reshape(R, NC // 128, 128)
              + offs[:, :, None]).reshape(R, NC)
    keep = gt | (eq & (prefix <= m_allow.astype(jnp.float32)))
    o_ref[...] = jnp.where(keep, sim, 0.0)


def _similarity_topk(emb, embt):
    return pl.pallas_call(
        _p4_body,
        grid=(_N // _R4,),
        in_specs=[
            pl.BlockSpec((_R4, _OSIZE), lambda i: (i, 0)),
            pl.BlockSpec((_OSIZE, _N), lambda i: (0, 0)),
        ],
        out_specs=pl.BlockSpec((_R4, _N), lambda i: (i, 0)),
        out_shape=jax.ShapeDtypeStruct((_N, _N), jnp.float32),
    )(emb, embt)


def kernel(features, adj, W1a, b1a, W2a, b2a, W1b, b1b, W2b, b2b):
    h = jax.nn.relu(features @ W1a + b1a)
    h = adj @ h
    h = h @ W2a + b2a
    h = jax.nn.relu(h)
    h2 = jax.nn.relu(h @ W1b + b1b)
    g2 = adj @ h2
    e = g2 @ W2b + b2b
    norm = jnp.linalg.norm(e, axis=1, keepdims=True)
    emb = e / jnp.maximum(norm, 1e-12)
    embt = emb.T
    return _similarity_topk(emb, embt)


# speedup vs baseline: 1.6760x; 1.0428x over previous
import jax
import jax.numpy as jnp
from jax.experimental import pallas as pl

_N = 8192
_HID = 32
_OSIZE = 128
_K = 30
_EPS = 0.1
_R4 = 256


def _p4_body(emb_ref, embt_ref, o_ref):
    sim = jnp.dot(emb_ref[...], embt_ref[...],
                  preferred_element_type=jnp.float32)
    R, NC = sim.shape
    cand = sim > _EPS
    work = jnp.where(cand, sim, -1.0)

    def cond(st):
        _, removed = st
        return jnp.any(removed < (_K + 1))

    def body(st):
        t, removed = st
        m = jnp.max(jnp.where(work < t, work, -1.0), axis=1, keepdims=True)
        cum = jnp.sum((work >= m).astype(jnp.int32), axis=1, keepdims=True)
        active = removed < (_K + 1)
        t = jnp.where(active, m, t)
        removed = jnp.where(active, cum, removed)
        return (t, removed)

    t0 = jnp.full((R, 1), 2.0, jnp.float32)
    r0 = jnp.zeros((R, 1), jnp.int32)
    t, _ = jax.lax.while_loop(cond, body, (t0, r0))

    gt = work > t
    c_gt = jnp.sum(gt.astype(jnp.int32), axis=1, keepdims=True)
    m_allow = (_K + 1) - c_gt
    eq = cand & (sim == t)
    eqf = eq.astype(jnp.float32)
    eq2 = eqf.reshape(R * (NC // 128), 128)
    i0 = jax.lax.broadcasted_iota(jnp.int32, (128, 128), 0)
    i1 = jax.lax.broadcasted_iota(jnp.int32, (128, 128), 1)
    tri = (i0 <= i1).astype(jnp.float32)
    within = jnp.dot(eq2, tri, preferred_element_type=jnp.float32)
    ctot = within[:, 127:128].reshape(R, NC // 128)
    j0 = jax.lax.broadcasted_iota(jnp.int32, (NC // 128, NC // 128), 0)
    j1 = jax.lax.broadcasted_iota(jnp.int32, (NC // 128, NC // 128), 1)
    stri = (j0 < j1).astype(jnp.float32)
    offs = jnp.dot(ctot, stri, preferred_element_type=jnp.float32)
    prefix = (within.reshape(R, NC // 128, 128)
              + offs[:, :, None]).reshape(R, NC)
    keep = gt | (eq & (prefix <= m_allow.astype(jnp.float32)))
    o_ref[...] = jnp.where(keep, sim, 0.0)


def _similarity_topk(emb, embt):
    return pl.pallas_call(
        _p4_body,
        grid=(_N // _R4,),
        in_specs=[
            pl.BlockSpec((_R4, _OSIZE), lambda i: (i, 0)),
            pl.BlockSpec((_OSIZE, _N), lambda i: (0, 0)),
        ],
        out_specs=pl.BlockSpec((_R4, _N), lambda i: (i, 0)),
        out_shape=jax.ShapeDtypeStruct((_N, _N), jnp.float32),
    )(emb, embt)


def kernel(features, adj, W1a, b1a, W2a, b2a, W1b, b1b, W2b, b2b):
    h = jax.nn.relu(features @ W1a + b1a)
    h = adj @ h
    h = h @ W2a + b2a
    h = jax.nn.relu(h)
    h2 = jax.nn.relu(h @ W1b + b1b)
    g2 = adj @ h2
    e = g2 @ W2b + b2b
    norm = jnp.linalg.norm(e, axis=1, keepdims=True)
    emb = e / jnp.maximum(norm, 1e-12)
    embt = emb.T
    return _similarity_topk(emb, embt)
